# Initial kernel scaffold; baseline (speedup 1.0000x reference)
import functools

import jax
import jax.numpy as jnp
from jax import lax
from jax.experimental import pallas as pl
from jax.experimental.pallas import tpu as pltpu
from jax.experimental.pallas import tpu_sc as plsc

N = 10000
E = 320000
D = 128
H = 128
C = 10

NC = 2
NS = 16
NW = NC * NS
CHUNK = 128
N_PAD = 10240
ROWS_PER_TILE = N_PAD // NS
CHUNKS = -(-E // (NW * CHUNK))
E_PAD = NW * CHUNKS * CHUNK
BL = 1024
GRID = N_PAD // BL

_mesh = plsc.VectorSubcoreMesh(core_axis_name="c", subcore_axis_name="s",
                               num_cores=NC)


@functools.partial(
    pl.kernel,
    mesh=_mesh,
    out_type=jax.ShapeDtypeStruct((NC, N_PAD, 16), jnp.float32),
    scratch_types=[
        pltpu.VMEM((CHUNK,), jnp.int32),
        pltpu.VMEM((CHUNK, 16), jnp.float32),
        pltpu.VMEM_SHARED((N_PAD, 16), jnp.float32),
    ],
)
def _sc_degree(dst_hbm, ones_hbm, zeros_hbm, out_hbm, dst_v, ones_v, acc_sh):
    c = lax.axis_index("c")
    s = lax.axis_index("s")
    w = c * NS + s
    pltpu.sync_copy(zeros_hbm, acc_sh.at[pl.ds(s * ROWS_PER_TILE, ROWS_PER_TILE)])
    pltpu.sync_copy(ones_hbm, ones_v)
    plsc.subcore_barrier()

    def step(i, carry):
        pltpu.sync_copy(dst_hbm.at[w, i], dst_v)
        pltpu.sync_copy(ones_v, acc_sh.at[dst_v], add=True)
        return carry

    lax.fori_loop(0, CHUNKS, step, 0)
    plsc.subcore_barrier()
    pltpu.sync_copy(acc_sh.at[pl.ds(s * ROWS_PER_TILE, ROWS_PER_TILE)],
                    out_hbm.at[c, pl.ds(s * ROWS_PER_TILE, ROWS_PER_TILE)])


@functools.partial(
    pl.kernel,
    mesh=_mesh,
    out_type=jax.ShapeDtypeStruct((NC, N_PAD, H), jnp.float32),
    scratch_types=[
        pltpu.VMEM((CHUNK,), jnp.int32),
        pltpu.VMEM((CHUNK,), jnp.int32),
        pltpu.VMEM((CHUNK, H), jnp.float32),
        pltpu.VMEM_SHARED((N_PAD, H), jnp.float32),
        pltpu.SemaphoreType.DMA,
    ],
)
def _sc_conv(u_hbm, src_hbm, dst_hbm, zeros_hbm, out_hbm,
             src_v, dst_v, rows_v, acc_sh, sem):
    c = lax.axis_index("c")
    s = lax.axis_index("s")
    w = c * NS + s
    pltpu.sync_copy(zeros_hbm, acc_sh.at[pl.ds(s * ROWS_PER_TILE, ROWS_PER_TILE)])
    plsc.subcore_barrier()

    def step(i, carry):
        pltpu.sync_copy(src_hbm.at[w, i], src_v)
        pltpu.sync_copy(dst_hbm.at[w, i], dst_v)
        pltpu.async_copy(u_hbm.at[src_v], rows_v, sem).wait()
        pltpu.sync_copy(rows_v, acc_sh.at[dst_v], add=True)
        return carry

    lax.fori_loop(0, CHUNKS, step, 0)
    plsc.subcore_barrier()
    pltpu.sync_copy(acc_sh.at[pl.ds(s * ROWS_PER_TILE, ROWS_PER_TILE)],
                    out_hbm.at[c, pl.ds(s * ROWS_PER_TILE, ROWS_PER_TILE)])


def _rows(i):
    return i * BL + lax.broadcasted_iota(jnp.int32, (BL, 1), 0)


def _deg_dinv(degp_ref):
    deg = degp_ref[0, :, 0:1] + degp_ref[1, :, 0:1]
    dinv = 1.0 / jnp.sqrt(jnp.maximum(deg, 1.0))
    return deg, dinv


def _encode_body(x_ref, degp_ref, wx_ref, ws_ref, b_ref, u_ref):
    i = pl.program_id(0)
    deg, dinv = _deg_dinv(degp_ref)
    struct = jnp.log(deg + 1.0)
    h = (jnp.dot(x_ref[...], wx_ref[...], preferred_element_type=jnp.float32)
         + struct * ws_ref[...] + b_ref[...])
    u_ref[...] = jnp.where(_rows(i) < N, h * dinv, 0.0)


def _conv_body(sp_ref, degp_ref, w_ref, b_ref, u_ref, h_ref):
    i = pl.program_id(0)
    _, dinv = _deg_dinv(degp_ref)
    agg = (sp_ref[0] + sp_ref[1]) * dinv
    h = jnp.maximum(
        jnp.dot(agg, w_ref[...], preferred_element_type=jnp.float32) + b_ref[...],
        0.0)
    h = jnp.where(_rows(i) < N, h, 0.0)
    h_ref[...] = h
    u_ref[...] = h * dinv


def _final_body(h_ref, g_ref, be_ref, wh_ref, bh_ref, out_ref, acc_ref):
    i = pl.program_id(0)

    @pl.when(i == 0)
    def _():
        acc_ref[...] = jnp.zeros_like(acc_ref)

    h = h_ref[...]
    mu = jnp.mean(h, axis=1, keepdims=True)
    var = jnp.mean((h - mu) ** 2, axis=1, keepdims=True)
    hn = (h - mu) / jnp.sqrt(var + 1e-5) * g_ref[...] + be_ref[...]
    hn = jnp.where(_rows(i) < N, hn, 0.0)
    acc_ref[...] += jnp.sum(hn, axis=0, keepdims=True)

    @pl.when(i == GRID - 1)
    def _():
        g = acc_ref[...] * (1.0 / N)
        out_ref[...] = (jnp.dot(g, wh_ref[...], preferred_element_type=jnp.float32)
                        + bh_ref[...])


_row_spec = pl.BlockSpec((BL, H), lambda i: (i, 0))
_degp_spec = pl.BlockSpec((NC, BL, 16), lambda i: (0, i, 0))
_sp_spec = pl.BlockSpec((NC, BL, H), lambda i: (0, i, 0))
_w_spec = pl.BlockSpec((H, H), lambda i: (0, 0))
_b_spec = pl.BlockSpec((1, H), lambda i: (0, 0))

_encode_call = pl.pallas_call(
    _encode_body,
    grid=(GRID,),
    in_specs=[_row_spec, _degp_spec, _w_spec, _b_spec, _b_spec],
    out_specs=_row_spec,
    out_shape=jax.ShapeDtypeStruct((N_PAD, H), jnp.float32),
)

_conv_call = pl.pallas_call(
    _conv_body,
    grid=(GRID,),
    in_specs=[_sp_spec, _degp_spec, _w_spec, _b_spec],
    out_specs=[_row_spec, _row_spec],
    out_shape=[jax.ShapeDtypeStruct((N_PAD, H), jnp.float32),
               jax.ShapeDtypeStruct((N_PAD, H), jnp.float32)],
)

_final_call = pl.pallas_call(
    _final_body,
    grid=(GRID,),
    in_specs=[_row_spec, _b_spec, _b_spec, _w_spec, _b_spec],
    out_specs=pl.BlockSpec((1, H), lambda i: (0, 0)),
    out_shape=jax.ShapeDtypeStruct((1, H), jnp.float32),
    scratch_shapes=[pltpu.VMEM((1, H), jnp.float32)],
)


def kernel(x, edge_index, W_enc, b_enc, W_c0, b_c0, W_c1, b_c1, W_c2, b_c2,
           gamma, beta, W_head, b_head):
    src = edge_index[0].astype(jnp.int32)
    dst = edge_index[1].astype(jnp.int32)
    pad_e = E_PAD - E
    src_r = jnp.pad(src, (0, pad_e), constant_values=N_PAD - 1).reshape(
        NW, CHUNKS, CHUNK)
    dst_r = jnp.pad(dst, (0, pad_e), constant_values=N_PAD - 1).reshape(
        NW, CHUNKS, CHUNK)
    x_p = jnp.pad(x, ((0, N_PAD - N), (0, 0)))
    zeros16 = jnp.zeros((ROWS_PER_TILE, 16), jnp.float32)
    zerosH = jnp.zeros((ROWS_PER_TILE, H), jnp.float32)
    ones16 = jnp.ones((CHUNK, 16), jnp.float32)
    wx = W_enc[:D]
    ws = W_enc[D:D + 1]
    b_enc2 = b_enc.reshape(1, H)
    gamma2 = gamma.reshape(1, H)
    beta2 = beta.reshape(1, H)
    wh = jnp.pad(W_head, ((0, 0), (0, H - C)))
    bh = jnp.pad(b_head, (0, H - C)).reshape(1, H)

    degp = _sc_degree(dst_r, ones16, zeros16)

    u = _encode_call(x_p, degp, wx, ws, b_enc2)

    for W, b in ((W_c0, b_c0), (W_c1, b_c1), (W_c2, b_c2)):
        sp = _sc_conv(u, src_r, dst_r, zerosH)
        u, h = _conv_call(sp, degp, W, b.reshape(1, H))

    out = _final_call(h, gamma2, beta2, wh, bh)
    return out[:, :C]


# speedup vs baseline: 5.9119x; 5.9119x over previous
import functools

import jax
import jax.numpy as jnp
from jax import lax
from jax.experimental import pallas as pl
from jax.experimental.pallas import tpu as pltpu
from jax.experimental.pallas import tpu_sc as plsc

N = 10000
E = 320000
D = 128
H = 128
C = 10

NC = 2
NS = 16
NW = NC * NS
CHUNK = 128
N_PAD = 10240
ROWS_PER_TILE = N_PAD // NS
CHUNKS = -(-E // (NW * CHUNK))
E_PAD = NW * CHUNKS * CHUNK
BL = 1024
GRID = N_PAD // BL


@functools.cache
def _make_sc_conv():
  mesh = plsc.VectorSubcoreMesh(core_axis_name="c", subcore_axis_name="s",
                                num_cores=NC)

  @functools.partial(
      pl.kernel,
      mesh=mesh,
      out_type=jax.ShapeDtypeStruct((NC, N_PAD, H), jnp.float32),
      scratch_types=[
          pltpu.VMEM((CHUNK,), jnp.int32),
          pltpu.VMEM((CHUNK,), jnp.int32),
          pltpu.VMEM((CHUNK, H), jnp.float32),
          pltpu.VMEM_SHARED((N_PAD, H), jnp.float32),
          pltpu.SemaphoreType.DMA,
      ],
  )
  def _sc_conv(u_hbm, src_hbm, dst_hbm, zeros_hbm, out_hbm,
               src_v, dst_v, rows_v, acc_sh, sem):
    c = lax.axis_index("c")
    s = lax.axis_index("s")
    w = c * NS + s
    pltpu.sync_copy(zeros_hbm, acc_sh.at[pl.ds(s * ROWS_PER_TILE, ROWS_PER_TILE)])
    plsc.subcore_barrier()

    def step(i, carry):
        pltpu.sync_copy(src_hbm.at[w, i], src_v)
        pltpu.sync_copy(dst_hbm.at[w, i], dst_v)
        pltpu.async_copy(u_hbm.at[src_v], rows_v, sem).wait()
        pltpu.sync_copy(rows_v, acc_sh.at[dst_v], add=True)
        return carry

    lax.fori_loop(0, CHUNKS, step, 0)
    plsc.subcore_barrier()
    pltpu.sync_copy(acc_sh.at[pl.ds(s * ROWS_PER_TILE, ROWS_PER_TILE)],
                    out_hbm.at[c, pl.ds(s * ROWS_PER_TILE, ROWS_PER_TILE)])

  return _sc_conv


def _rows(i):
    return i * BL + lax.broadcasted_iota(jnp.int32, (BL, 1), 0)


def _deg_dinv(degp_ref):
    deg = degp_ref[0, :, 0:1] + degp_ref[1, :, 0:1]
    dinv = 1.0 / jnp.sqrt(jnp.maximum(deg, 1.0))
    return deg, dinv


def _encode_body(x_ref, degp_ref, wx_ref, ws_ref, b_ref, u_ref):
    i = pl.program_id(0)
    deg, dinv = _deg_dinv(degp_ref)
    struct = jnp.log(deg + 1.0)
    h = (jnp.dot(x_ref[...], wx_ref[...], preferred_element_type=jnp.float32)
         + struct * ws_ref[...] + b_ref[...])
    u_ref[...] = jnp.where(_rows(i) < N, h * dinv, 0.0)


def _conv_body(sp_ref, degp_ref, w_ref, b_ref, u_ref, h_ref):
    i = pl.program_id(0)
    _, dinv = _deg_dinv(degp_ref)
    agg = (sp_ref[0] + sp_ref[1]) * dinv
    h = jnp.maximum(
        jnp.dot(agg, w_ref[...], preferred_element_type=jnp.float32) + b_ref[...],
        0.0)
    h = jnp.where(_rows(i) < N, h, 0.0)
    h_ref[...] = h
    u_ref[...] = h * dinv


def _final_body(h_ref, g_ref, be_ref, wh_ref, bh_ref, out_ref, acc_ref):
    i = pl.program_id(0)

    @pl.when(i == 0)
    def _():
        acc_ref[...] = jnp.zeros_like(acc_ref)

    h = h_ref[...]
    mu = jnp.mean(h, axis=1, keepdims=True)
    var = jnp.mean((h - mu) ** 2, axis=1, keepdims=True)
    hn = (h - mu) / jnp.sqrt(var + 1e-5) * g_ref[...] + be_ref[...]
    hn = jnp.where(_rows(i) < N, hn, 0.0)
    acc_ref[...] += jnp.sum(hn, axis=0, keepdims=True)

    @pl.when(i == GRID - 1)
    def _():
        g = acc_ref[...] * (1.0 / N)
        out_ref[...] = (jnp.dot(g, wh_ref[...], preferred_element_type=jnp.float32)
                        + bh_ref[...])


_row_spec = pl.BlockSpec((BL, H), lambda i: (i, 0))
_degp_spec = pl.BlockSpec((NC, BL, H), lambda i: (0, i, 0))
_sp_spec = pl.BlockSpec((NC, BL, H), lambda i: (0, i, 0))
_w_spec = pl.BlockSpec((H, H), lambda i: (0, 0))
_b_spec = pl.BlockSpec((1, H), lambda i: (0, 0))

_encode_call = pl.pallas_call(
    _encode_body,
    grid=(GRID,),
    in_specs=[_row_spec, _degp_spec, _w_spec, _b_spec, _b_spec],
    out_specs=_row_spec,
    out_shape=jax.ShapeDtypeStruct((N_PAD, H), jnp.float32),
)

_conv_call = pl.pallas_call(
    _conv_body,
    grid=(GRID,),
    in_specs=[_sp_spec, _degp_spec, _w_spec, _b_spec],
    out_specs=[_row_spec, _row_spec],
    out_shape=[jax.ShapeDtypeStruct((N_PAD, H), jnp.float32),
               jax.ShapeDtypeStruct((N_PAD, H), jnp.float32)],
)

_final_call = pl.pallas_call(
    _final_body,
    grid=(GRID,),
    in_specs=[_row_spec, _b_spec, _b_spec, _w_spec, _b_spec],
    out_specs=pl.BlockSpec((1, H), lambda i: (0, 0)),
    out_shape=jax.ShapeDtypeStruct((1, H), jnp.float32),
    scratch_shapes=[pltpu.VMEM((1, H), jnp.float32)],
)


def kernel(x, edge_index, W_enc, b_enc, W_c0, b_c0, W_c1, b_c1, W_c2, b_c2,
           gamma, beta, W_head, b_head):
    src = edge_index[0].astype(jnp.int32)
    dst = edge_index[1].astype(jnp.int32)
    pad_e = E_PAD - E
    src_r = jnp.pad(src, (0, pad_e), constant_values=N_PAD - 1).reshape(
        NW, CHUNKS, CHUNK)
    dst_r = jnp.pad(dst, (0, pad_e), constant_values=N_PAD - 1).reshape(
        NW, CHUNKS, CHUNK)
    x_p = jnp.pad(x, ((0, N_PAD - N), (0, 0)))
    zerosH = jnp.zeros((ROWS_PER_TILE, H), jnp.float32)
    row_ids = lax.broadcasted_iota(jnp.int32, (N_PAD, 1), 0)
    ones_mat = jnp.where(row_ids < N, 1.0, 0.0) * jnp.ones((1, H), jnp.float32)
    wx = W_enc[:D]
    ws = W_enc[D:D + 1]
    b_enc2 = b_enc.reshape(1, H)
    gamma2 = gamma.reshape(1, H)
    beta2 = beta.reshape(1, H)
    wh = jnp.pad(W_head, ((0, 0), (0, H - C)))
    bh = jnp.pad(b_head, (0, H - C)).reshape(1, H)

    degp = _make_sc_conv()(ones_mat, src_r, dst_r, zerosH)

    u = _encode_call(x_p, degp, wx, ws, b_enc2)

    for W, b in ((W_c0, b_c0), (W_c1, b_c1), (W_c2, b_c2)):
        sp = _make_sc_conv()(u, src_r, dst_r, zerosH)
        u, h = _conv_call(sp, degp, W, b.reshape(1, H))

    out = _final_call(h, gamma2, beta2, wh, bh)
    return out[:, :C]
